# Initial kernel scaffold; baseline (speedup 1.0000x reference)
import jax
import jax.numpy as jnp
from jax.experimental import pallas as pl


def kernel(seq, query, target, embed, in_proj_w, in_proj_b, attn_out_w, attn_out_b, ff_w1, ff_b1, ff_w2, ff_b2, norm1_g, norm1_b, norm2_g, norm2_b, gate_w, gate_b, q_embed, qp_w, qp_b, op_w, op_b):
    raise NotImplementedError("write your pallas kernel here")


# speedup vs baseline: 1.8490x; 1.8490x over previous
import functools

import jax
import jax.numpy as jnp
from jax.experimental import pallas as pl
from jax.experimental.pallas import tpu as pltpu

HIDDEN_DIM = 64
MEMORY_SLOTS = 8
VOCAB_SIZE = 64
N_HEADS = 2
HEAD_DIM = HIDDEN_DIM // N_HEADS
B = 128
L = 512
BB = 8

_TRANS_RHS = (((1,), (1,)), ((), ()))


def _dot(a, b):
    return jax.lax.dot_general(a, b, (((1,), (0,)), ((), ())),
                               preferred_element_type=jnp.float32)


def _dot_tb(a, b):
    return jax.lax.dot_general(a, b, _TRANS_RHS,
                               preferred_element_type=jnp.float32)


def _layer_norm(x, g, b):
    m = jnp.mean(x, axis=-1, keepdims=True)
    v = jnp.mean((x - m) ** 2, axis=-1, keepdims=True)
    return (x - m) * jax.lax.rsqrt(v + 1e-5) * g + b


def _step(seq_ref, query_ref, target_ref, embed_ref,
          wq0_ref, wq1_ref, wk0_ref, wk1_ref, wv0_ref, wv1_ref,
          bq0_ref, bq1_ref, bk0_ref, bk1_ref, bv0_ref, bv1_ref,
          wo0_ref, wo1_ref, bo_ref, w1_ref, b1_ref, w2_ref, b2_ref,
          n1g_ref, n1b_ref, n2g_ref, n2b_ref, gate_ref,
          qemb_ref, wqp_ref, bqp_ref, wop_ref, bop_ref, out_ref):
    i = pl.program_id(0)

    @pl.when(i == 0)
    def _():
        out_ref[...] = jnp.zeros_like(out_ref)

    BL = BB * L
    iota_col = jax.lax.broadcasted_iota(jnp.int32, (L, 1), 0)
    iota_row = jax.lax.broadcasted_iota(jnp.int32, (1, L), 1)
    iota_v = jax.lax.broadcasted_iota(jnp.int32, (BL, VOCAB_SIZE), 1)
    iota_v_row = jax.lax.broadcasted_iota(jnp.int32, (1, VOCAB_SIZE), 1)

    onehot = (seq_ref[...] == iota_v).astype(jnp.float32)
    h0 = _dot(onehot, embed_ref[...])

    q0 = _dot(h0, wq0_ref[...]) + bq0_ref[...]
    k0 = _dot(h0, wk0_ref[...]) + bk0_ref[...]
    v0 = _dot(h0, wv0_ref[...]) + bv0_ref[...]
    q1 = _dot(h0, wq1_ref[...]) + bq1_ref[...]
    k1 = _dot(h0, wk1_ref[...]) + bk1_ref[...]
    v1 = _dot(h0, wv1_ref[...]) + bv1_ref[...]

    scale = 1.0 / (HEAD_DIM ** 0.5)

    def att_head(qh, kh, vh):
        s = _dot_tb(qh, kh) * scale
        m = jnp.max(s, axis=1, keepdims=True)
        e = jnp.exp(s - m)
        p = e / jnp.sum(e, axis=1, keepdims=True)
        return _dot(p, vh)

    a0_parts = []
    a1_parts = []
    for b in range(BB):
        sl = slice(b * L, (b + 1) * L)
        a0_parts.append(att_head(q0[sl], k0[sl], v0[sl]))
        a1_parts.append(att_head(q1[sl], k1[sl], v1[sl]))
    a0 = jnp.concatenate(a0_parts, axis=0)
    a1 = jnp.concatenate(a1_parts, axis=0)
    a_out = _dot(a0, wo0_ref[...]) + _dot(a1, wo1_ref[...]) + bo_ref[...]

    h1 = _layer_norm(h0 + a_out, n1g_ref[...], n1b_ref[...])
    ff = _dot(jnp.maximum(_dot(h1, w1_ref[...]) + b1_ref[...], 0.0),
              w2_ref[...]) + b2_ref[...]
    h2 = _layer_norm(h1 + ff, n2g_ref[...], n2b_ref[...])

    s_all = jnp.sum(h2 * gate_ref[...], axis=1, keepdims=True)

    acc = jnp.float32(0.0)
    for b in range(BB):
        sl = slice(b * L, (b + 1) * L)
        s_col = s_all[sl]
        rows = []
        for _k in range(MEMORY_SLOTS):
            mx = jnp.max(s_col)
            cand = jnp.where(s_col == mx, iota_col, jnp.int32(2 ** 30))
            idx = jnp.min(cand)
            rows.append((iota_row == idx).astype(jnp.float32))
            s_col = jnp.where(iota_col == idx, jnp.float32(-1e30), s_col)
        sel = jnp.concatenate(rows, axis=0)
        mem = _dot(sel, h2[sl])

        q_idx = query_ref[i * BB + b]
        q_oh = (iota_v_row == q_idx).astype(jnp.float32)
        q_h = _dot(q_oh, qemb_ref[...])
        qp = _dot(q_h, wqp_ref[...]) + bqp_ref[...]
        s2 = jnp.sum(mem * qp, axis=1, keepdims=True) * (1.0 / (HIDDEN_DIM ** 0.5))
        m2 = jnp.max(s2)
        e2 = jnp.exp(s2 - m2)
        w = e2 / jnp.sum(e2)
        read = jnp.sum(w * mem, axis=0, keepdims=True)

        logits = _dot(read, wop_ref[...]) + bop_ref[...]
        ml = jnp.max(logits)
        lse = ml + jnp.log(jnp.sum(jnp.exp(logits - ml)))
        t_idx = target_ref[i * BB + b]
        t_oh = (iota_v_row == t_idx).astype(jnp.float32)
        tgt = jnp.sum(logits * t_oh)
        acc = acc + (lse - tgt)

    out_ref[...] += acc * (1.0 / B)


@functools.partial(jax.jit, static_argnames=("interpret",))
def _run(seq, query, target, embed, in_proj_w, in_proj_b, attn_out_w,
         attn_out_b, ff_w1, ff_b1, ff_w2, ff_b2, norm1_g, norm1_b, norm2_g,
         norm2_b, gate_w, gate_b, q_embed, qp_w, qp_b, op_w, op_b,
         interpret=False):
    f32 = jnp.float32
    seq_col = seq.astype(jnp.int32).reshape(B * L, 1)
    query = query.astype(jnp.int32)
    target = target.astype(jnp.int32)
    HD = HEAD_DIM
    wq0 = in_proj_w[0:HD].T
    wq1 = in_proj_w[HD:2 * HD].T
    wk0 = in_proj_w[2 * HD:3 * HD].T
    wk1 = in_proj_w[3 * HD:4 * HD].T
    wv0 = in_proj_w[4 * HD:5 * HD].T
    wv1 = in_proj_w[5 * HD:6 * HD].T
    bq0 = in_proj_b[0:HD].reshape(1, HD)
    bq1 = in_proj_b[HD:2 * HD].reshape(1, HD)
    bk0 = in_proj_b[2 * HD:3 * HD].reshape(1, HD)
    bk1 = in_proj_b[3 * HD:4 * HD].reshape(1, HD)
    bv0 = in_proj_b[4 * HD:5 * HD].reshape(1, HD)
    bv1 = in_proj_b[5 * HD:6 * HD].reshape(1, HD)
    wo0 = attn_out_w.T[0:HD]
    wo1 = attn_out_w.T[HD:2 * HD]
    bo = attn_out_b.reshape(1, HIDDEN_DIM)
    w1 = ff_w1.T
    b1 = ff_b1.reshape(1, -1)
    w2 = ff_w2.T
    b2 = ff_b2.reshape(1, -1)
    n1g = norm1_g.reshape(1, -1)
    n1b = norm1_b.reshape(1, -1)
    n2g = norm2_g.reshape(1, -1)
    n2b = norm2_b.reshape(1, -1)
    gate = gate_w.reshape(1, -1)
    wqp = qp_w.T
    bqp = qp_b.reshape(1, -1)
    wop = op_w.T
    bop = op_b.reshape(1, -1)

    full = lambda a: pl.BlockSpec(a.shape, lambda i: (0,) * a.ndim)
    smem = pl.BlockSpec(memory_space=pltpu.SMEM)
    vm_args = (embed, wq0, wq1, wk0, wk1, wv0, wv1, bq0, bq1, bk0, bk1,
               bv0, bv1, wo0, wo1, bo, w1, b1, w2, b2, n1g, n1b, n2g, n2b,
               gate, q_embed, wqp, bqp, wop, bop)
    out = pl.pallas_call(
        _step,
        grid=(B // BB,),
        in_specs=[pl.BlockSpec((BB * L, 1), lambda i: (i, 0)), smem, smem]
                 + [full(a) for a in vm_args],
        out_specs=pl.BlockSpec((1, 1), lambda i: (0, 0)),
        out_shape=jax.ShapeDtypeStruct((1, 1), f32),
        interpret=interpret,
    )(seq_col, query, target, *vm_args)
    return out[0, 0]


def kernel(seq, query, target, embed, in_proj_w, in_proj_b, attn_out_w,
           attn_out_b, ff_w1, ff_b1, ff_w2, ff_b2, norm1_g, norm1_b, norm2_g,
           norm2_b, gate_w, gate_b, q_embed, qp_w, qp_b, op_w, op_b):
    return _run(seq, query, target, embed, in_proj_w, in_proj_b, attn_out_w,
                attn_out_b, ff_w1, ff_b1, ff_w2, ff_b2, norm1_g, norm1_b,
                norm2_g, norm2_b, gate_w, gate_b, q_embed, qp_w, qp_b,
                op_w, op_b)
